# Initial kernel scaffold; baseline (speedup 1.0000x reference)
import jax
import jax.numpy as jnp
from jax.experimental import pallas as pl


def kernel(x, hyperedge_index, jaccard_edge_index, jaccard_edge_weight, node_batch_idx, W, b):
    raise NotImplementedError("write your pallas kernel here")


# speedup vs baseline: 3.0346x; 3.0346x over previous
import functools

import jax
import jax.numpy as jnp
from jax import lax
from jax.experimental import pallas as pl
from jax.experimental.pallas import tpu as pltpu
from jax.experimental.pallas import tpu_sc as plsc

NN = 10000
NE = 10000
EMB = 128
DH = 64
NI = 320000
NJ = 200000
NG = 64
NC = 2
NS = 16
K = 80

NCH_I = NI // K
NCH_J = NJ // K
NCH_R = NN // K


def _matmul_body(x_ref, w_ref, out_ref):
    m = jnp.dot(x_ref[...], w_ref[...], preferred_element_type=jnp.float32)
    out_ref[0] = m[:, :DH]
    out_ref[1] = m[:, DH:]


def _matmul_halves(x, W):
    return pl.pallas_call(
        _matmul_body,
        grid=(10,),
        in_specs=[
            pl.BlockSpec((1000, EMB), lambda i: (i, 0)),
            pl.BlockSpec((EMB, EMB), lambda i: (0, 0)),
        ],
        out_specs=pl.BlockSpec((2, 1000, DH), lambda i: (0, i, 0)),
        out_shape=jax.ShapeDtypeStruct((2, NN, DH), jnp.float32),
    )(x, W)


def _sc_body(m2, he_n, he_e, ji_s, ji_d, jw, out, idx1_v, idx2_v, w_v, cnt_v,
             ones_v, rows_v, acc_a, acc_b, bcnt, dcnt):
    c = lax.axis_index("c")
    s = lax.axis_index("s")
    rbase = c * NN

    trips_i = (NCH_I - 1 - s) // NS + 1
    trips_j = (NCH_J - 1 - s) // NS + 1
    trips_r = (NCH_R - 1 - s) // NS + 1

    def _zero_rows(r, _):
        for k in range(DH // 16):
            rows_v[r, pl.ds(k * 16, 16)] = jnp.zeros((16,), jnp.float32)
        return 0

    def _load_a(t, _):
        off = (s + NS * t) * K
        pltpu.sync_copy(m2.at[pl.ds(rbase + off, K), :], acc_a.at[pl.ds(off, K), :])
        return 0
    lax.fori_loop(0, trips_r, _load_a, 0)

    lax.fori_loop(0, K, _zero_rows, 0)
    for k in range(K // 16):
        ones_v[pl.ds(k * 16, 16)] = jnp.ones((16,), jnp.float32)
        cnt_v[pl.ds(k * 16, 16)] = jnp.zeros((16,), jnp.float32)

    def _zero_b(t, _):
        off = (s + NS * t) * K
        pltpu.sync_copy(rows_v, acc_b.at[pl.ds(off, K), :])
        pltpu.sync_copy(cnt_v, bcnt.at[pl.ds(off, K)])
        pltpu.sync_copy(cnt_v, dcnt.at[pl.ds(off, K)])
        return 0
    lax.fori_loop(0, trips_r, _zero_b, 0)

    plsc.subcore_barrier()

    def _stage1(t, _):
        off = (s + NS * t) * K
        pltpu.sync_copy(he_n.at[pl.ds(off, K)], idx1_v)
        pltpu.sync_copy(he_e.at[pl.ds(off, K)], idx2_v)
        pltpu.sync_copy(acc_a.at[idx1_v], rows_v)
        pltpu.sync_copy(rows_v, acc_b.at[idx2_v], add=True)
        pltpu.sync_copy(ones_v, bcnt.at[idx2_v], add=True)
        pltpu.sync_copy(ones_v, dcnt.at[idx1_v], add=True)
        return 0
    lax.fori_loop(0, trips_i, _stage1, 0)

    plsc.subcore_barrier()

    def _scale_rows_by(vec_ref, recip):
        def _grp(g, _):
            base = g * 16
            v = vec_ref[pl.ds(base, 16)]
            if recip:
                v = 1.0 / jnp.maximum(v, 1.0)
            for j in range(16):
                sc = v[j]
                for k in range(DH // 16):
                    sl = pl.ds(k * 16, 16)
                    rows_v[base + j, sl] = rows_v[base + j, sl] * sc
            return 0
        lax.fori_loop(0, K // 16, _grp, 0)

    def _norm_e(t, _):
        off = (s + NS * t) * K
        pltpu.sync_copy(acc_b.at[pl.ds(off, K), :], rows_v)
        pltpu.sync_copy(bcnt.at[pl.ds(off, K)], cnt_v)
        _scale_rows_by(cnt_v, True)
        pltpu.sync_copy(rows_v, acc_b.at[pl.ds(off, K), :])
        pltpu.sync_copy(rows_v, acc_a.at[pl.ds(off, K), :])
        return 0
    lax.fori_loop(0, trips_r, _norm_e, 0)

    plsc.subcore_barrier()

    def _stage2(t, _):
        off = (s + NS * t) * K
        pltpu.sync_copy(ji_s.at[pl.ds(off, K)], idx1_v)
        pltpu.sync_copy(ji_d.at[pl.ds(off, K)], idx2_v)
        pltpu.sync_copy(jw.at[pl.ds(off, K)], w_v)
        pltpu.sync_copy(acc_b.at[idx1_v], rows_v)
        _scale_rows_by(w_v, False)
        pltpu.sync_copy(rows_v, acc_a.at[idx2_v], add=True)
        return 0
    lax.fori_loop(0, trips_j, _stage2, 0)

    plsc.subcore_barrier()

    lax.fori_loop(0, K, _zero_rows, 0)

    def _zero_b2(t, _):
        off = (s + NS * t) * K
        pltpu.sync_copy(rows_v, acc_b.at[pl.ds(off, K), :])
        return 0
    lax.fori_loop(0, trips_r, _zero_b2, 0)

    plsc.subcore_barrier()

    def _stage3(t, _):
        off = (s + NS * t) * K
        pltpu.sync_copy(he_n.at[pl.ds(off, K)], idx1_v)
        pltpu.sync_copy(he_e.at[pl.ds(off, K)], idx2_v)
        pltpu.sync_copy(acc_a.at[idx2_v], rows_v)
        pltpu.sync_copy(rows_v, acc_b.at[idx1_v], add=True)
        return 0
    lax.fori_loop(0, trips_i, _stage3, 0)

    plsc.subcore_barrier()

    def _fin(t, _):
        off = (s + NS * t) * K
        pltpu.sync_copy(acc_b.at[pl.ds(off, K), :], rows_v)
        pltpu.sync_copy(dcnt.at[pl.ds(off, K)], cnt_v)
        _scale_rows_by(cnt_v, True)
        pltpu.sync_copy(rows_v, out.at[pl.ds(rbase + off, K), :])
        return 0
    lax.fori_loop(0, trips_r, _fin, 0)


@functools.partial(jax.jit, static_argnames=())
def _sc_pipeline(m2_flat, he_n, he_e, ji_s, ji_d, jw):
    mesh = plsc.VectorSubcoreMesh(
        core_axis_name="c", subcore_axis_name="s", num_cores=NC, num_subcores=NS)
    return pl.kernel(
        _sc_body,
        out_type=jax.ShapeDtypeStruct((NC * NN, DH), jnp.float32),
        mesh=mesh,
        compiler_params=pltpu.CompilerParams(use_tc_tiling_on_sc=False),
        scratch_types=[
            pltpu.VMEM((K,), jnp.int32),
            pltpu.VMEM((K,), jnp.int32),
            pltpu.VMEM((K,), jnp.float32),
            pltpu.VMEM((K,), jnp.float32),
            pltpu.VMEM((K,), jnp.float32),
            pltpu.VMEM((K, DH), jnp.float32),
            pltpu.VMEM_SHARED((NN, DH), jnp.float32),
            pltpu.VMEM_SHARED((NN, DH), jnp.float32),
            pltpu.VMEM_SHARED((NN,), jnp.float32),
            pltpu.VMEM_SHARED((NN,), jnp.float32),
        ],
    )(m2_flat, he_n, he_e, ji_s, ji_d, jw)


def _finish_body(out2_ref, b_ref, nbi_ref, z_ref, zg_ref, zgacc, cntacc):
    i = pl.program_id(0)
    o = jnp.concatenate([out2_ref[0], out2_ref[1]], axis=-1) + b_ref[...]
    c0 = jnp.sqrt(2.0 / jnp.pi).astype(jnp.float32)
    z = 0.5 * o * (1.0 + jnp.tanh(c0 * (o + 0.044715 * (o * o * o))))
    z_ref[...] = z

    nbi = nbi_ref[0, 0]
    gid = lax.broadcasted_iota(jnp.int32, (NG, 1000), 0)
    onehot = (gid == nbi[None, :]).astype(jnp.float32)

    zg_part = jnp.dot(onehot, z, preferred_element_type=jnp.float32)
    cnt_part = jnp.sum(onehot, axis=1, keepdims=True)

    @pl.when(i == 0)
    def _():
        zgacc[...] = jnp.zeros_like(zgacc)
        cntacc[...] = jnp.zeros_like(cntacc)

    zgacc[...] += zg_part
    cntacc[...] += cnt_part

    @pl.when(i == 9)
    def _():
        zg_ref[...] = zgacc[...] / jnp.maximum(cntacc[...], 1.0)


def _finish(out2, b2d, nbi3):
    return pl.pallas_call(
        _finish_body,
        grid=(10,),
        in_specs=[
            pl.BlockSpec((2, 1000, DH), lambda i: (0, i, 0)),
            pl.BlockSpec((1, EMB), lambda i: (0, 0)),
            pl.BlockSpec((1, 1, 1000), lambda i: (i, 0, 0)),
        ],
        out_specs=[
            pl.BlockSpec((1000, EMB), lambda i: (i, 0)),
            pl.BlockSpec((NG, EMB), lambda i: (0, 0)),
        ],
        out_shape=[
            jax.ShapeDtypeStruct((NN, EMB), jnp.float32),
            jax.ShapeDtypeStruct((NG, EMB), jnp.float32),
        ],
        scratch_shapes=[
            pltpu.VMEM((NG, EMB), jnp.float32),
            pltpu.VMEM((NG, 1), jnp.float32),
        ],
    )(out2, b2d, nbi3)


def kernel(x, hyperedge_index, jaccard_edge_index, jaccard_edge_weight,
           node_batch_idx, W, b):
    m2 = _matmul_halves(x, W)
    out_flat = _sc_pipeline(m2.reshape(NC * NN, DH),
                            hyperedge_index[0], hyperedge_index[1],
                            jaccard_edge_index[0], jaccard_edge_index[1],
                            jaccard_edge_weight)
    out2 = out_flat.reshape(NC, NN, DH)
    z, z_graph = _finish(out2, b.reshape(1, EMB),
                         node_batch_idx.reshape(10, 1, 1000))
    return (z, z_graph)


# speedup vs baseline: 6.3521x; 2.0932x over previous
import functools

import jax
import jax.numpy as jnp
from jax import lax
from jax.experimental import pallas as pl
from jax.experimental.pallas import tpu as pltpu
from jax.experimental.pallas import tpu_sc as plsc

NN = 10000
NE = 10000
EMB = 128
DH = 64
NI = 320000
NJ = 200000
NG = 64
NC = 2
NS = 16
K = 80

NCH_I = NI // K
NCH_J = NJ // K
NCH_R = NN // K

NB = 3
IDXB = K * 4
ROWB = K * DH * 4


def _matmul_body(x_ref, w_ref, out_ref):
    m = jnp.dot(x_ref[...], w_ref[...], preferred_element_type=jnp.float32)
    out_ref[0] = m[:, :DH]
    out_ref[1] = m[:, DH:]


def _matmul_halves(x, W):
    return pl.pallas_call(
        _matmul_body,
        grid=(10,),
        in_specs=[
            pl.BlockSpec((1000, EMB), lambda i: (i, 0)),
            pl.BlockSpec((EMB, EMB), lambda i: (0, 0)),
        ],
        out_specs=pl.BlockSpec((2, 1000, DH), lambda i: (0, i, 0)),
        out_shape=jax.ShapeDtypeStruct((2, NN, DH), jnp.float32),
    )(x, W)


def _sc_body(m2, he_n, he_e, ji_s, ji_d, jw, out, idx1, idx2, wbuf, cnt_v,
             ones_v, rows, sem_i1, sem_i2, sem_w, sem_g, sem_sc, sem_cb,
             sem_cd, acc_a, acc_b, bcnt, dcnt):
    c = lax.axis_index("c")
    s = lax.axis_index("s")
    rbase = c * NN

    trips_i = (NCH_I - 1 - s) // NS + 1
    trips_j = (NCH_J - 1 - s) // NS + 1
    trips_r = (NCH_R - 1 - s) // NS + 1

    def _off(t):
        return (s + NS * t) * K

    def _zero_rows0(r, _):
        for k in range(DH // 16):
            rows[0, r, pl.ds(k * 16, 16)] = jnp.zeros((16,), jnp.float32)
        return 0

    def _load_a(t, _):
        off = _off(t)
        pltpu.sync_copy(m2.at[pl.ds(rbase + off, K), :], acc_a.at[pl.ds(off, K), :])
        return 0
    lax.fori_loop(0, trips_r, _load_a, 0)

    lax.fori_loop(0, K, _zero_rows0, 0)
    for k in range(K // 16):
        ones_v[pl.ds(k * 16, 16)] = jnp.ones((16,), jnp.float32)
        cnt_v[pl.ds(k * 16, 16)] = jnp.zeros((16,), jnp.float32)

    def _zero_b(t, _):
        off = _off(t)
        pltpu.sync_copy(rows.at[0], acc_b.at[pl.ds(off, K), :])
        pltpu.sync_copy(cnt_v, bcnt.at[pl.ds(off, K)])
        pltpu.sync_copy(cnt_v, dcnt.at[pl.ds(off, K)])
        return 0
    lax.fori_loop(0, trips_r, _zero_b, 0)

    plsc.subcore_barrier()

    def _scale_rows_by(p, vec_ref, recip):
        def _grp(g, _):
            base = g * 16
            v = vec_ref[pl.ds(base, 16)]
            if recip:
                v = 1.0 / jnp.maximum(v, 1.0)
            for j in range(16):
                sc = v[j]
                for k in range(DH // 16):
                    sl = pl.ds(k * 16, 16)
                    rows[p, base + j, sl] = rows[p, base + j, sl] * sc
            return 0
        lax.fori_loop(0, K // 16, _grp, 0)

    def _run_stage(trips, start_idx, wait_idx, start_gather, process,
                   start_scatter, wait_scatter):
        start_idx(0, 0)
        n_iters = (trips + NB - 1) // NB

        def _body(i, _):
            for p in range(NB):
                t = i * NB + p
                p1 = (p + 1) % NB

                @pl.when(t < trips)
                def _():
                    wait_idx(p)
                    gdesc = start_gather(p)

                    @pl.when(t + 1 < trips)
                    def _():
                        @pl.when(t >= NB - 1)
                        def _():
                            wait_scatter(p1)
                        start_idx(p1, t + 1)

                    gdesc.wait()
                    process(p)
                    start_scatter(p)
            return 0
        lax.fori_loop(0, n_iters, _body, 0)
        for p in range(NB):
            wait_scatter(p)

    def _s1_start_idx(p, t):
        off = _off(t)
        pltpu.make_async_copy(he_n.at[pl.ds(off, K)], idx1.at[p], sem_i1.at[p]).start()
        pltpu.make_async_copy(he_e.at[pl.ds(off, K)], idx2.at[p], sem_i2.at[p]).start()

    def _s1_wait_idx(p):
        pltpu.make_async_copy(he_n.at[pl.ds(0, K)], idx1.at[p], sem_i1.at[p]).wait()
        pltpu.make_async_copy(he_e.at[pl.ds(0, K)], idx2.at[p], sem_i2.at[p]).wait()

    def _s1_start_gather(p):
        d = pltpu.make_async_copy(acc_a.at[idx1.at[p]], rows.at[p], sem_g.at[p])
        d.start()
        return d

    def _s1_start_scatter(p):
        pltpu.async_copy(rows.at[p], acc_b.at[idx2.at[p]], sem_sc.at[p], add=True)
        pltpu.async_copy(ones_v, bcnt.at[idx2.at[p]], sem_cb.at[p], add=True)
        pltpu.async_copy(ones_v, dcnt.at[idx1.at[p]], sem_cd.at[p], add=True)

    def _s1_wait_scatter(p):
        pltpu.make_async_copy(m2.at[pl.ds(0, K), :], rows.at[p], sem_sc.at[p]).wait()
        pltpu.make_async_copy(jw.at[pl.ds(0, K)], wbuf.at[p], sem_cb.at[p]).wait()
        pltpu.make_async_copy(jw.at[pl.ds(0, K)], wbuf.at[p], sem_cd.at[p]).wait()

    _run_stage(trips_i, _s1_start_idx, _s1_wait_idx, _s1_start_gather,
               lambda p: None, _s1_start_scatter, _s1_wait_scatter)

    plsc.subcore_barrier()

    def _norm_e(t, _):
        off = _off(t)
        pltpu.sync_copy(acc_b.at[pl.ds(off, K), :], rows.at[0])
        pltpu.sync_copy(bcnt.at[pl.ds(off, K)], cnt_v)
        _scale_rows_by(0, cnt_v, True)
        pltpu.sync_copy(rows.at[0], acc_b.at[pl.ds(off, K), :])
        pltpu.sync_copy(rows.at[0], acc_a.at[pl.ds(off, K), :])
        return 0
    lax.fori_loop(0, trips_r, _norm_e, 0)

    plsc.subcore_barrier()

    def _s2_start_idx(p, t):
        off = _off(t)
        pltpu.make_async_copy(ji_s.at[pl.ds(off, K)], idx1.at[p], sem_i1.at[p]).start()
        pltpu.make_async_copy(ji_d.at[pl.ds(off, K)], idx2.at[p], sem_i2.at[p]).start()
        pltpu.make_async_copy(jw.at[pl.ds(off, K)], wbuf.at[p], sem_w.at[p]).start()

    def _s2_wait_idx(p):
        pltpu.make_async_copy(ji_s.at[pl.ds(0, K)], idx1.at[p], sem_i1.at[p]).wait()
        pltpu.make_async_copy(ji_d.at[pl.ds(0, K)], idx2.at[p], sem_i2.at[p]).wait()
        pltpu.make_async_copy(jw.at[pl.ds(0, K)], wbuf.at[p], sem_w.at[p]).wait()

    def _s2_start_gather(p):
        d = pltpu.make_async_copy(acc_b.at[idx1.at[p]], rows.at[p], sem_g.at[p])
        d.start()
        return d

    def _s2_process(p):
        _scale_rows_by(p, wbuf.at[p], False)

    def _s2_start_scatter(p):
        pltpu.async_copy(rows.at[p], acc_a.at[idx2.at[p]], sem_sc.at[p], add=True)

    def _s2_wait_scatter(p):
        pltpu.make_async_copy(m2.at[pl.ds(0, K), :], rows.at[p], sem_sc.at[p]).wait()

    _run_stage(trips_j, _s2_start_idx, _s2_wait_idx, _s2_start_gather,
               _s2_process, _s2_start_scatter, _s2_wait_scatter)

    plsc.subcore_barrier()

    lax.fori_loop(0, K, _zero_rows0, 0)

    def _zero_b2(t, _):
        off = _off(t)
        pltpu.sync_copy(rows.at[0], acc_b.at[pl.ds(off, K), :])
        return 0
    lax.fori_loop(0, trips_r, _zero_b2, 0)

    plsc.subcore_barrier()

    def _s3_start_gather(p):
        d = pltpu.make_async_copy(acc_a.at[idx2.at[p]], rows.at[p], sem_g.at[p])
        d.start()
        return d

    def _s3_start_scatter(p):
        pltpu.async_copy(rows.at[p], acc_b.at[idx1.at[p]], sem_sc.at[p], add=True)

    def _s3_wait_scatter(p):
        pltpu.make_async_copy(m2.at[pl.ds(0, K), :], rows.at[p], sem_sc.at[p]).wait()

    _run_stage(trips_i, _s1_start_idx, _s1_wait_idx, _s3_start_gather,
               lambda p: None, _s3_start_scatter, _s3_wait_scatter)

    plsc.subcore_barrier()

    def _fin(t, _):
        off = _off(t)
        pltpu.sync_copy(acc_b.at[pl.ds(off, K), :], rows.at[0])
        pltpu.sync_copy(dcnt.at[pl.ds(off, K)], cnt_v)
        _scale_rows_by(0, cnt_v, True)
        pltpu.sync_copy(rows.at[0], out.at[pl.ds(rbase + off, K), :])
        return 0
    lax.fori_loop(0, trips_r, _fin, 0)


@functools.partial(jax.jit, static_argnames=())
def _sc_pipeline(m2_flat, he_n, he_e, ji_s, ji_d, jw):
    mesh = plsc.VectorSubcoreMesh(
        core_axis_name="c", subcore_axis_name="s", num_cores=NC, num_subcores=NS)
    return pl.kernel(
        _sc_body,
        out_type=jax.ShapeDtypeStruct((NC * NN, DH), jnp.float32),
        mesh=mesh,
        compiler_params=pltpu.CompilerParams(use_tc_tiling_on_sc=False),
        scratch_types=[
            pltpu.VMEM((NB, K), jnp.int32),
            pltpu.VMEM((NB, K), jnp.int32),
            pltpu.VMEM((NB, K), jnp.float32),
            pltpu.VMEM((K,), jnp.float32),
            pltpu.VMEM((K,), jnp.float32),
            pltpu.VMEM((NB, K, DH), jnp.float32),
            pltpu.SemaphoreType.DMA((NB,)),
            pltpu.SemaphoreType.DMA((NB,)),
            pltpu.SemaphoreType.DMA((NB,)),
            pltpu.SemaphoreType.DMA((NB,)),
            pltpu.SemaphoreType.DMA((NB,)),
            pltpu.SemaphoreType.DMA((NB,)),
            pltpu.SemaphoreType.DMA((NB,)),
            pltpu.VMEM_SHARED((NN, DH), jnp.float32),
            pltpu.VMEM_SHARED((NN, DH), jnp.float32),
            pltpu.VMEM_SHARED((NN,), jnp.float32),
            pltpu.VMEM_SHARED((NN,), jnp.float32),
        ],
    )(m2_flat, he_n, he_e, ji_s, ji_d, jw)


def _finish_body(out2_ref, b_ref, nbi_ref, z_ref, zg_ref, zgacc, cntacc):
    i = pl.program_id(0)
    o = jnp.concatenate([out2_ref[0], out2_ref[1]], axis=-1) + b_ref[...]
    c0 = jnp.sqrt(2.0 / jnp.pi).astype(jnp.float32)
    z = 0.5 * o * (1.0 + jnp.tanh(c0 * (o + 0.044715 * (o * o * o))))
    z_ref[...] = z

    nbi = nbi_ref[0, 0]
    gid = lax.broadcasted_iota(jnp.int32, (NG, 1000), 0)
    onehot = (gid == nbi[None, :]).astype(jnp.float32)

    zg_part = jnp.dot(onehot, z, preferred_element_type=jnp.float32)
    cnt_part = jnp.sum(onehot, axis=1, keepdims=True)

    @pl.when(i == 0)
    def _():
        zgacc[...] = jnp.zeros_like(zgacc)
        cntacc[...] = jnp.zeros_like(cntacc)

    zgacc[...] += zg_part
    cntacc[...] += cnt_part

    @pl.when(i == 9)
    def _():
        zg_ref[...] = zgacc[...] / jnp.maximum(cntacc[...], 1.0)


def _finish(out2, b2d, nbi3):
    return pl.pallas_call(
        _finish_body,
        grid=(10,),
        in_specs=[
            pl.BlockSpec((2, 1000, DH), lambda i: (0, i, 0)),
            pl.BlockSpec((1, EMB), lambda i: (0, 0)),
            pl.BlockSpec((1, 1, 1000), lambda i: (i, 0, 0)),
        ],
        out_specs=[
            pl.BlockSpec((1000, EMB), lambda i: (i, 0)),
            pl.BlockSpec((NG, EMB), lambda i: (0, 0)),
        ],
        out_shape=[
            jax.ShapeDtypeStruct((NN, EMB), jnp.float32),
            jax.ShapeDtypeStruct((NG, EMB), jnp.float32),
        ],
        scratch_shapes=[
            pltpu.VMEM((NG, EMB), jnp.float32),
            pltpu.VMEM((NG, 1), jnp.float32),
        ],
    )(out2, b2d, nbi3)


def kernel(x, hyperedge_index, jaccard_edge_index, jaccard_edge_weight,
           node_batch_idx, W, b):
    m2 = _matmul_halves(x, W)
    out_flat = _sc_pipeline(m2.reshape(NC * NN, DH),
                            hyperedge_index[0], hyperedge_index[1],
                            jaccard_edge_index[0], jaccard_edge_index[1],
                            jaccard_edge_weight)
    out2 = out_flat.reshape(NC, NN, DH)
    z, z_graph = _finish(out2, b.reshape(1, EMB),
                         node_batch_idx.reshape(10, 1, 1000))
    return (z, z_graph)


# speedup vs baseline: 6.3859x; 1.0053x over previous
import functools

import jax
import jax.numpy as jnp
from jax import lax
from jax.experimental import pallas as pl
from jax.experimental.pallas import tpu as pltpu
from jax.experimental.pallas import tpu_sc as plsc

NN = 10000
NE = 10000
EMB = 128
DH = 64
NI = 320000
NJ = 200000
NG = 64
NC = 2
NS = 16
K = 80

NCH_I = NI // K
NCH_J = NJ // K
NCH_R = NN // K

NB = 3
IDXB = K * 4
ROWB = K * DH * 4


def _matmul_body(x_ref, w_ref, out_ref):
    m = jnp.dot(x_ref[...], w_ref[...], preferred_element_type=jnp.float32)
    out_ref[0] = m[:, :DH]
    out_ref[1] = m[:, DH:]


def _matmul_halves(x, W):
    return pl.pallas_call(
        _matmul_body,
        grid=(10,),
        in_specs=[
            pl.BlockSpec((1000, EMB), lambda i: (i, 0)),
            pl.BlockSpec((EMB, EMB), lambda i: (0, 0)),
        ],
        out_specs=pl.BlockSpec((2, 1000, DH), lambda i: (0, i, 0)),
        out_shape=jax.ShapeDtypeStruct((2, NN, DH), jnp.float32),
    )(x, W)


def _sc_body(m2, he_n, he_e, ji_s, ji_d, jw, out, idx1, idx2, wbuf, cnt_v,
             ones_v, rows, sem_i1, sem_i2, sem_w, sem_g, sem_sc, sem_cb,
             sem_cd, acc_a, acc_b, bcnt, dcnt):
    c = lax.axis_index("c")
    s = lax.axis_index("s")
    rbase = c * NN

    trips_i = (NCH_I - 1 - s) // NS + 1
    trips_j = (NCH_J - 1 - s) // NS + 1
    trips_r = (NCH_R - 1 - s) // NS + 1

    def _off(t):
        return (s + NS * t) * K

    def _zero_rows0(r, _):
        for k in range(DH // 16):
            rows[0, r, pl.ds(k * 16, 16)] = jnp.zeros((16,), jnp.float32)
        return 0

    def _load_a(t, _):
        off = _off(t)
        pltpu.sync_copy(m2.at[pl.ds(rbase + off, K), :], acc_a.at[pl.ds(off, K), :])
        return 0
    lax.fori_loop(0, trips_r, _load_a, 0)

    lax.fori_loop(0, K, _zero_rows0, 0)
    for k in range(K // 16):
        ones_v[pl.ds(k * 16, 16)] = jnp.ones((16,), jnp.float32)
        cnt_v[pl.ds(k * 16, 16)] = jnp.zeros((16,), jnp.float32)

    def _zero_b(t, _):
        off = _off(t)
        pltpu.sync_copy(rows.at[0], acc_b.at[pl.ds(off, K), :])
        pltpu.sync_copy(cnt_v, bcnt.at[pl.ds(off, K)])
        pltpu.sync_copy(cnt_v, dcnt.at[pl.ds(off, K)])
        return 0
    lax.fori_loop(0, trips_r, _zero_b, 0)

    plsc.subcore_barrier()

    def _scale_rows_by(p, vec_ref, recip):
        def _grp(g, _):
            base = g * 16
            v = vec_ref[pl.ds(base, 16)]
            if recip:
                v = 1.0 / jnp.maximum(v, 1.0)
            for j in range(16):
                sc = v[j]
                for k in range(DH // 16):
                    sl = pl.ds(k * 16, 16)
                    rows[p, base + j, sl] = rows[p, base + j, sl] * sc
            return 0
        lax.fori_loop(0, K // 16, _grp, 0)

    def _run_stage(trips, start_idx, wait_idx, start_gather, process,
                   start_scatter, wait_scatter, wait_gather=None):
        if wait_gather is None:
            def wait_gather(p):
                pltpu.make_async_copy(m2.at[pl.ds(0, K), :], rows.at[p],
                                      sem_g.at[p]).wait()
        start_idx(0, 0)

        @pl.when(trips > 1)
        def _():
            start_idx(1, 1)
        wait_idx(0)
        start_gather(0)
        n_iters = (trips + NB - 1) // NB

        def _body(i, _):
            for p in range(NB):
                t = i * NB + p
                p1 = (p + 1) % NB
                p2 = (p + 2) % NB

                @pl.when(t < trips)
                def _():
                    wait_gather(p)

                    @pl.when(t + 2 < trips)
                    def _():
                        @pl.when(t >= 1)
                        def _():
                            wait_scatter(p2)
                        start_idx(p2, t + 2)

                    @pl.when(t + 1 < trips)
                    def _():
                        wait_idx(p1)
                        start_gather(p1)

                    process(p)
                    start_scatter(p)
            return 0
        lax.fori_loop(0, n_iters, _body, 0)
        for p in range(NB):
            wait_scatter(p)

    def _s1_start_idx(p, t):
        off = _off(t)
        pltpu.make_async_copy(he_n.at[pl.ds(off, K)], idx1.at[p], sem_i1.at[p]).start()
        pltpu.make_async_copy(he_e.at[pl.ds(off, K)], idx2.at[p], sem_i2.at[p]).start()

    def _s1_wait_idx(p):
        pltpu.make_async_copy(he_n.at[pl.ds(0, K)], idx1.at[p], sem_i1.at[p]).wait()
        pltpu.make_async_copy(he_e.at[pl.ds(0, K)], idx2.at[p], sem_i2.at[p]).wait()

    def _s1_start_gather(p):
        d = pltpu.make_async_copy(acc_a.at[idx1.at[p]], rows.at[p], sem_g.at[p])
        d.start()
        return d

    def _s1_start_scatter(p):
        pltpu.async_copy(rows.at[p], acc_b.at[idx2.at[p]], sem_sc.at[p], add=True)
        pltpu.async_copy(ones_v, bcnt.at[idx2.at[p]], sem_cb.at[p], add=True)
        pltpu.async_copy(ones_v, dcnt.at[idx1.at[p]], sem_cd.at[p], add=True)

    def _s1_wait_scatter(p):
        pltpu.make_async_copy(m2.at[pl.ds(0, K), :], rows.at[p], sem_sc.at[p]).wait()
        pltpu.make_async_copy(jw.at[pl.ds(0, K)], wbuf.at[p], sem_cb.at[p]).wait()
        pltpu.make_async_copy(jw.at[pl.ds(0, K)], wbuf.at[p], sem_cd.at[p]).wait()

    _run_stage(trips_i, _s1_start_idx, _s1_wait_idx, _s1_start_gather,
               lambda p: None, _s1_start_scatter, _s1_wait_scatter)

    plsc.subcore_barrier()

    def _norm_e(t, _):
        off = _off(t)
        pltpu.sync_copy(acc_b.at[pl.ds(off, K), :], rows.at[0])
        pltpu.sync_copy(bcnt.at[pl.ds(off, K)], cnt_v)
        _scale_rows_by(0, cnt_v, True)
        pltpu.sync_copy(rows.at[0], acc_b.at[pl.ds(off, K), :])
        pltpu.sync_copy(rows.at[0], acc_a.at[pl.ds(off, K), :])
        return 0
    lax.fori_loop(0, trips_r, _norm_e, 0)

    plsc.subcore_barrier()

    def _s2_start_idx(p, t):
        off = _off(t)
        pltpu.make_async_copy(ji_s.at[pl.ds(off, K)], idx1.at[p], sem_i1.at[p]).start()
        pltpu.make_async_copy(ji_d.at[pl.ds(off, K)], idx2.at[p], sem_i2.at[p]).start()
        pltpu.make_async_copy(jw.at[pl.ds(off, K)], wbuf.at[p], sem_w.at[p]).start()

    def _s2_wait_idx(p):
        pltpu.make_async_copy(ji_s.at[pl.ds(0, K)], idx1.at[p], sem_i1.at[p]).wait()
        pltpu.make_async_copy(ji_d.at[pl.ds(0, K)], idx2.at[p], sem_i2.at[p]).wait()
        pltpu.make_async_copy(jw.at[pl.ds(0, K)], wbuf.at[p], sem_w.at[p]).wait()

    def _s2_start_gather(p):
        d = pltpu.make_async_copy(acc_b.at[idx1.at[p]], rows.at[p], sem_g.at[p])
        d.start()
        return d

    def _s2_process(p):
        _scale_rows_by(p, wbuf.at[p], False)

    def _s2_start_scatter(p):
        pltpu.async_copy(rows.at[p], acc_a.at[idx2.at[p]], sem_sc.at[p], add=True)

    def _s2_wait_scatter(p):
        pltpu.make_async_copy(m2.at[pl.ds(0, K), :], rows.at[p], sem_sc.at[p]).wait()

    _run_stage(trips_j, _s2_start_idx, _s2_wait_idx, _s2_start_gather,
               _s2_process, _s2_start_scatter, _s2_wait_scatter)

    plsc.subcore_barrier()

    lax.fori_loop(0, K, _zero_rows0, 0)

    def _zero_b2(t, _):
        off = _off(t)
        pltpu.sync_copy(rows.at[0], acc_b.at[pl.ds(off, K), :])
        return 0
    lax.fori_loop(0, trips_r, _zero_b2, 0)

    plsc.subcore_barrier()

    def _s3_start_gather(p):
        d = pltpu.make_async_copy(acc_a.at[idx2.at[p]], rows.at[p], sem_g.at[p])
        d.start()
        return d

    def _s3_start_scatter(p):
        pltpu.async_copy(rows.at[p], acc_b.at[idx1.at[p]], sem_sc.at[p], add=True)

    def _s3_wait_scatter(p):
        pltpu.make_async_copy(m2.at[pl.ds(0, K), :], rows.at[p], sem_sc.at[p]).wait()

    _run_stage(trips_i, _s1_start_idx, _s1_wait_idx, _s3_start_gather,
               lambda p: None, _s3_start_scatter, _s3_wait_scatter)

    plsc.subcore_barrier()

    def _fin(t, _):
        off = _off(t)
        pltpu.sync_copy(acc_b.at[pl.ds(off, K), :], rows.at[0])
        pltpu.sync_copy(dcnt.at[pl.ds(off, K)], cnt_v)
        _scale_rows_by(0, cnt_v, True)
        pltpu.sync_copy(rows.at[0], out.at[pl.ds(rbase + off, K), :])
        return 0
    lax.fori_loop(0, trips_r, _fin, 0)


@functools.partial(jax.jit, static_argnames=())
def _sc_pipeline(m2_flat, he_n, he_e, ji_s, ji_d, jw):
    mesh = plsc.VectorSubcoreMesh(
        core_axis_name="c", subcore_axis_name="s", num_cores=NC, num_subcores=NS)
    return pl.kernel(
        _sc_body,
        out_type=jax.ShapeDtypeStruct((NC * NN, DH), jnp.float32),
        mesh=mesh,
        compiler_params=pltpu.CompilerParams(use_tc_tiling_on_sc=False),
        scratch_types=[
            pltpu.VMEM((NB, K), jnp.int32),
            pltpu.VMEM((NB, K), jnp.int32),
            pltpu.VMEM((NB, K), jnp.float32),
            pltpu.VMEM((K,), jnp.float32),
            pltpu.VMEM((K,), jnp.float32),
            pltpu.VMEM((NB, K, DH), jnp.float32),
            pltpu.SemaphoreType.DMA((NB,)),
            pltpu.SemaphoreType.DMA((NB,)),
            pltpu.SemaphoreType.DMA((NB,)),
            pltpu.SemaphoreType.DMA((NB,)),
            pltpu.SemaphoreType.DMA((NB,)),
            pltpu.SemaphoreType.DMA((NB,)),
            pltpu.SemaphoreType.DMA((NB,)),
            pltpu.VMEM_SHARED((NN, DH), jnp.float32),
            pltpu.VMEM_SHARED((NN, DH), jnp.float32),
            pltpu.VMEM_SHARED((NN,), jnp.float32),
            pltpu.VMEM_SHARED((NN,), jnp.float32),
        ],
    )(m2_flat, he_n, he_e, ji_s, ji_d, jw)


def _finish_body(out2_ref, b_ref, nbi_ref, z_ref, zg_ref, zgacc, cntacc):
    i = pl.program_id(0)
    o = jnp.concatenate([out2_ref[0], out2_ref[1]], axis=-1) + b_ref[...]
    c0 = jnp.sqrt(2.0 / jnp.pi).astype(jnp.float32)
    z = 0.5 * o * (1.0 + jnp.tanh(c0 * (o + 0.044715 * (o * o * o))))
    z_ref[...] = z

    nbi = nbi_ref[0, 0]
    gid = lax.broadcasted_iota(jnp.int32, (NG, 1000), 0)
    onehot = (gid == nbi[None, :]).astype(jnp.float32)

    zg_part = jnp.dot(onehot, z, preferred_element_type=jnp.float32)
    cnt_part = jnp.sum(onehot, axis=1, keepdims=True)

    @pl.when(i == 0)
    def _():
        zgacc[...] = jnp.zeros_like(zgacc)
        cntacc[...] = jnp.zeros_like(cntacc)

    zgacc[...] += zg_part
    cntacc[...] += cnt_part

    @pl.when(i == 9)
    def _():
        zg_ref[...] = zgacc[...] / jnp.maximum(cntacc[...], 1.0)


def _finish(out2, b2d, nbi3):
    return pl.pallas_call(
        _finish_body,
        grid=(10,),
        in_specs=[
            pl.BlockSpec((2, 1000, DH), lambda i: (0, i, 0)),
            pl.BlockSpec((1, EMB), lambda i: (0, 0)),
            pl.BlockSpec((1, 1, 1000), lambda i: (i, 0, 0)),
        ],
        out_specs=[
            pl.BlockSpec((1000, EMB), lambda i: (i, 0)),
            pl.BlockSpec((NG, EMB), lambda i: (0, 0)),
        ],
        out_shape=[
            jax.ShapeDtypeStruct((NN, EMB), jnp.float32),
            jax.ShapeDtypeStruct((NG, EMB), jnp.float32),
        ],
        scratch_shapes=[
            pltpu.VMEM((NG, EMB), jnp.float32),
            pltpu.VMEM((NG, 1), jnp.float32),
        ],
    )(out2, b2d, nbi3)


def kernel(x, hyperedge_index, jaccard_edge_index, jaccard_edge_weight,
           node_batch_idx, W, b):
    m2 = _matmul_halves(x, W)
    out_flat = _sc_pipeline(m2.reshape(NC * NN, DH),
                            hyperedge_index[0], hyperedge_index[1],
                            jaccard_edge_index[0], jaccard_edge_index[1],
                            jaccard_edge_weight)
    out2 = out_flat.reshape(NC, NN, DH)
    z, z_graph = _finish(out2, b.reshape(1, EMB),
                         node_batch_idx.reshape(10, 1, 1000))
    return (z, z_graph)
